# Initial kernel scaffold; baseline (speedup 1.0000x reference)
import jax
import jax.numpy as jnp
from jax.experimental import pallas as pl


def kernel(x, neigh_x, kernel_self, kernel_neigh):
    raise NotImplementedError("write your pallas kernel here")


# speedup vs baseline: 1.3195x; 1.3195x over previous
import functools

import jax
import jax.numpy as jnp
from jax.experimental import pallas as pl

N = 10000
K = 32
D = 128
BN = 400


def _body(x_ref, nx_ref, ws_ref, wn_ref, o_ref):
    agg = jnp.sum(nx_ref[...], axis=1) * (1.0 / K)
    o_ref[...] = (
        jnp.dot(x_ref[...], ws_ref[...], preferred_element_type=jnp.float32)
        + jnp.dot(agg, wn_ref[...], preferred_element_type=jnp.float32)
    )


@functools.partial(jax.jit)
def kernel(x, neigh_x, kernel_self, kernel_neigh):
    grid = (N // BN,)
    return pl.pallas_call(
        _body,
        grid=grid,
        in_specs=[
            pl.BlockSpec((BN, D), lambda i: (i, 0)),
            pl.BlockSpec((BN, K, D), lambda i: (i, 0, 0)),
            pl.BlockSpec((D, D), lambda i: (0, 0)),
            pl.BlockSpec((D, D), lambda i: (0, 0)),
        ],
        out_specs=pl.BlockSpec((BN, D), lambda i: (i, 0)),
        out_shape=jax.ShapeDtypeStruct((N, D), jnp.float32),
    )(x, neigh_x, kernel_self, kernel_neigh)
